# baseline (speedup 1.0000x reference)
import functools

import jax
import jax.numpy as jnp
from jax import lax
from jax.experimental import pallas as pl
from jax.experimental.pallas import tpu as pltpu
from jax.experimental.pallas import tpu_sc as plsc

N = 10000
NPAD = 10240
DIN = 128
DOUT = 16
E = 160000
NC, NS = 2, 16
NW = NC * NS
CH = 128
CPW = 40
EPW = CH * CPW
EPAD = NW * EPW
SLC = NPAD // NS

@functools.cache
def _sc_mesh():
    return plsc.VectorSubcoreMesh(core_axis_name="c", subcore_axis_name="s",
                                  num_cores=NC, num_subcores=NS)


def _sc_deg_body(dst_ref, zeros_ref, ones_ref, out_ref, idxd_v, ones_v, sem,
                 deg_sh):
    cid = lax.axis_index("c")
    sid = lax.axis_index("s")
    wid = sid * NC + cid
    pltpu.sync_copy(ones_ref, ones_v)
    pltpu.sync_copy(dst_ref.at[pl.ds(wid * CPW, CPW)], idxd_v)
    pltpu.sync_copy(zeros_ref.at[pl.ds(sid * SLC, SLC)],
                    deg_sh.at[pl.ds(sid * SLC, SLC)])
    plsc.subcore_barrier()

    def fire(c, carry):
        pltpu.async_copy(ones_v, deg_sh.at[idxd_v.at[c]], sem, add=True)
        return carry

    lax.fori_loop(0, CPW, fire, 0)

    def drain(c, carry):
        pltpu.make_async_copy(ones_v, deg_sh.at[idxd_v.at[c]], sem).wait()
        return carry

    lax.fori_loop(0, CPW, drain, 0)
    plsc.subcore_barrier()
    pltpu.sync_copy(deg_sh.at[pl.ds(sid * SLC, SLC)],
                    out_ref.at[cid, pl.ds(sid * SLC, SLC)])


def _sc_deg(dst2d, zeros_n, ones_ch):
    return pl.kernel(
        _sc_deg_body,
        out_type=jax.ShapeDtypeStruct((NC, NPAD), jnp.float32),
        mesh=_sc_mesh(),
        compiler_params=pltpu.CompilerParams(use_tc_tiling_on_sc=False),
        scratch_types=[
            pltpu.VMEM((CPW, CH), jnp.int32),
            pltpu.VMEM((CH,), jnp.float32),
            pltpu.SemaphoreType.DMA,
            pltpu.VMEM_SHARED((NPAD,), jnp.float32),
        ],
    )(dst2d, zeros_n, ones_ch)


NBUF = 8


def _sc_agg_body(src_ref, dst_ref, hs_ref, zeros_ref, out_ref,
                 idxs_v, idxd_v, rows, sems, agg_sh, hs_sh):
    cid = lax.axis_index("c")
    sid = lax.axis_index("s")
    wid = sid * NC + cid
    pltpu.sync_copy(src_ref.at[pl.ds(wid * CPW, CPW)], idxs_v)
    pltpu.sync_copy(dst_ref.at[pl.ds(wid * CPW, CPW)], idxd_v)
    pltpu.sync_copy(zeros_ref.at[pl.ds(sid * SLC, SLC)],
                    agg_sh.at[pl.ds(sid * SLC, SLC)])
    pltpu.sync_copy(hs_ref.at[pl.ds(sid * SLC, SLC)],
                    hs_sh.at[pl.ds(sid * SLC, SLC)])
    plsc.subcore_barrier()

    for b in range(NBUF):
        pltpu.async_copy(hs_sh.at[idxs_v.at[b]], rows[b], sems[b])

    def body(g, carry):
        for b in range(NBUF):
            c = g * NBUF + b
            pltpu.make_async_copy(hs_sh.at[idxs_v.at[c]], rows[b],
                                  sems[b]).wait()
            pltpu.sync_copy(rows[b], agg_sh.at[idxd_v.at[c]], add=True)
            pltpu.async_copy(hs_sh.at[idxs_v.at[c + NBUF]], rows[b], sems[b])
        return carry

    lax.fori_loop(0, CPW // NBUF - 1, body, 0)
    for b in range(NBUF):
        c = CPW - NBUF + b
        pltpu.make_async_copy(hs_sh.at[idxs_v.at[c]], rows[b], sems[b]).wait()
        pltpu.sync_copy(rows[b], agg_sh.at[idxd_v.at[c]], add=True)

    plsc.subcore_barrier()
    pltpu.sync_copy(agg_sh.at[pl.ds(sid * SLC, SLC)],
                    out_ref.at[cid, pl.ds(sid * SLC, SLC)])


def _sc_agg(src2d, dst2d, hs, zeros_nd):
    return pl.kernel(
        _sc_agg_body,
        out_type=jax.ShapeDtypeStruct((NC, NPAD, DOUT), jnp.float32),
        mesh=_sc_mesh(),
        compiler_params=pltpu.CompilerParams(use_tc_tiling_on_sc=False),
        scratch_types=[
            pltpu.VMEM((CPW, CH), jnp.int32),
            pltpu.VMEM((CPW, CH), jnp.int32),
            [pltpu.VMEM((CH, DOUT), jnp.float32) for _ in range(NBUF)],
            [pltpu.SemaphoreType.DMA for _ in range(NBUF)],
            pltpu.VMEM_SHARED((NPAD, DOUT), jnp.float32),
            pltpu.VMEM_SHARED((NPAD, DOUT), jnp.float32),
        ],
    )(src2d, dst2d, hs, zeros_nd)


def _tc_h_body(x_ref, w_ref, h_ref):
    h = lax.dot_general(x_ref[...], w_ref[...], (((0,), (0,)), ((), ())),
                        preferred_element_type=jnp.float32)
    h_ref[0:N, :] = h
    h_ref[N:NPAD, :] = jnp.zeros((NPAD - N, DOUT), jnp.float32)


def _tc_h(x, w):
    return pl.pallas_call(
        _tc_h_body,
        out_shape=jax.ShapeDtypeStruct((NPAD, DOUT), jnp.float32),
    )(x, w)


def _tc_scale_body(h_ref, degt_ref, hs_ref):
    degsum = jnp.sum(degt_ref[...], axis=1, keepdims=True) + 1.0
    dinv = lax.rsqrt(degsum)
    hs_ref[...] = h_ref[...] * dinv


def _tc_scale(h, deg_t):
    return pl.pallas_call(
        _tc_scale_body,
        out_shape=jax.ShapeDtypeStruct((NPAD, DOUT), jnp.float32),
    )(h, deg_t)


def _tc_z_body(a0_ref, a1_ref, hs_ref, degt_ref, b_ref, z_ref):
    degsum = jnp.sum(degt_ref[...], axis=1, keepdims=True) + 1.0
    dinv = lax.rsqrt(degsum)
    z_ref[...] = dinv * (a0_ref[...] + a1_ref[...] + hs_ref[...]) + b_ref[...]


def _tc_z(a0, a1, hs, deg_t, b2):
    return pl.pallas_call(
        _tc_z_body,
        out_shape=jax.ShapeDtypeStruct((NPAD, DOUT), jnp.float32),
    )(a0, a1, hs, deg_t, b2)


BM = 2560
BN = 2048


def _tc_dec_body(z_ref, zt_ref, o_ref):
    d = lax.dot_general(z_ref[...], zt_ref[...], (((1,), (0,)), ((), ())),
                        preferred_element_type=jnp.float32)
    o_ref[...] = 0.5 + 0.5 * jnp.tanh(0.5 * d)


def _tc_decode(z, zt):
    return pl.pallas_call(
        _tc_dec_body,
        grid=(NPAD // BM, NPAD // BN),
        in_specs=[
            pl.BlockSpec((BM, DOUT), lambda i, j: (i, 0)),
            pl.BlockSpec((DOUT, BN), lambda i, j: (0, j)),
        ],
        out_specs=pl.BlockSpec((BM, BN), lambda i, j: (i, j)),
        out_shape=jax.ShapeDtypeStruct((N, N), jnp.float32),
        compiler_params=pltpu.CompilerParams(
            dimension_semantics=("parallel", "parallel")),
    )(z, zt)


@jax.jit
def kernel(x, edge_index, W_enc, b_enc):
    src = edge_index[0]
    dst = edge_index[1]
    pad = EPAD - E
    src_p = jnp.concatenate([src, jnp.zeros((pad,), jnp.int32)])
    dst_p = jnp.concatenate([dst, jnp.full((pad,), NPAD - 1, jnp.int32)])
    src2d = src_p.reshape(NW * CPW, CH)
    dst2d = dst_p.reshape(NW * CPW, CH)

    zeros_n = jnp.zeros((NPAD,), jnp.float32)
    ones_ch = jnp.ones((CH,), jnp.float32)
    zeros_nd = jnp.zeros((NPAD, DOUT), jnp.float32)

    deg_p = _sc_deg(dst2d, zeros_n, ones_ch)
    deg_t = deg_p.T

    h = _tc_h(x, W_enc)
    hs = _tc_scale(h, deg_t)

    agg_p = _sc_agg(src2d, dst2d, hs, zeros_nd)

    z = _tc_z(agg_p[0], agg_p[1], hs, deg_t,
              b_enc.reshape(1, DOUT))
    adj = _tc_decode(z, z.T)
    return adj


# speedup vs baseline: 1.0029x; 1.0029x over previous
import functools

import jax
import jax.numpy as jnp
from jax import lax
from jax.experimental import pallas as pl
from jax.experimental.pallas import tpu as pltpu
from jax.experimental.pallas import tpu_sc as plsc

N = 10000
NPAD = 10240
DIN = 128
DOUT = 16
E = 160000
NC, NS = 2, 16
NW = NC * NS
CH = 128
CPW = 40
EPW = CH * CPW
EPAD = NW * EPW
SLC = NPAD // NS

@functools.cache
def _sc_mesh():
    return plsc.VectorSubcoreMesh(core_axis_name="c", subcore_axis_name="s",
                                  num_cores=NC, num_subcores=NS)


def _sc_deg_body(dst_ref, zeros_ref, ones_ref, out_ref, idxd_v, ones_v, sem,
                 deg_sh):
    cid = lax.axis_index("c")
    sid = lax.axis_index("s")
    wid = sid * NC + cid
    pltpu.sync_copy(ones_ref, ones_v)
    pltpu.sync_copy(dst_ref.at[pl.ds(wid * CPW, CPW)], idxd_v)
    pltpu.sync_copy(zeros_ref.at[pl.ds(sid * SLC, SLC)],
                    deg_sh.at[pl.ds(sid * SLC, SLC)])
    plsc.subcore_barrier()

    def fire(c, carry):
        pltpu.async_copy(ones_v, deg_sh.at[idxd_v.at[c]], sem, add=True)
        return carry

    lax.fori_loop(0, CPW, fire, 0)

    def drain(c, carry):
        pltpu.make_async_copy(ones_v, deg_sh.at[idxd_v.at[c]], sem).wait()
        return carry

    lax.fori_loop(0, CPW, drain, 0)
    plsc.subcore_barrier()
    pltpu.sync_copy(deg_sh.at[pl.ds(sid * SLC, SLC)],
                    out_ref.at[cid, pl.ds(sid * SLC, SLC)])


def _sc_deg(dst2d, zeros_n, ones_ch):
    return pl.kernel(
        _sc_deg_body,
        out_type=jax.ShapeDtypeStruct((NC, NPAD), jnp.float32),
        mesh=_sc_mesh(),
        compiler_params=pltpu.CompilerParams(use_tc_tiling_on_sc=False),
        scratch_types=[
            pltpu.VMEM((CPW, CH), jnp.int32),
            pltpu.VMEM((CH,), jnp.float32),
            pltpu.SemaphoreType.DMA,
            pltpu.VMEM_SHARED((NPAD,), jnp.float32),
        ],
    )(dst2d, zeros_n, ones_ch)


NBUF = 8


def _sc_agg_body(src_ref, dst_ref, hs_ref, zeros_ref, out_ref,
                 idxs_v, idxd_v, rows, sems, agg_sh, hs_sh):
    cid = lax.axis_index("c")
    sid = lax.axis_index("s")
    wid = sid * NC + cid
    pltpu.sync_copy(src_ref.at[pl.ds(wid * CPW, CPW)], idxs_v)
    pltpu.sync_copy(dst_ref.at[pl.ds(wid * CPW, CPW)], idxd_v)
    pltpu.sync_copy(zeros_ref.at[pl.ds(sid * SLC, SLC)],
                    agg_sh.at[pl.ds(sid * SLC, SLC)])
    pltpu.sync_copy(hs_ref.at[pl.ds(sid * SLC, SLC)],
                    hs_sh.at[pl.ds(sid * SLC, SLC)])
    plsc.subcore_barrier()

    for b in range(NBUF):
        pltpu.async_copy(hs_sh.at[idxs_v.at[b]], rows[b], sems[b])

    def body(g, carry):
        for b in range(NBUF):
            c = g * NBUF + b
            pltpu.make_async_copy(hs_sh.at[idxs_v.at[c]], rows[b],
                                  sems[b]).wait()
            pltpu.sync_copy(rows[b], agg_sh.at[idxd_v.at[c]], add=True)
            pltpu.async_copy(hs_sh.at[idxs_v.at[c + NBUF]], rows[b], sems[b])
        return carry

    lax.fori_loop(0, CPW // NBUF - 1, body, 0)
    for b in range(NBUF):
        c = CPW - NBUF + b
        pltpu.make_async_copy(hs_sh.at[idxs_v.at[c]], rows[b], sems[b]).wait()
        pltpu.sync_copy(rows[b], agg_sh.at[idxd_v.at[c]], add=True)

    plsc.subcore_barrier()
    pltpu.sync_copy(agg_sh.at[pl.ds(sid * SLC, SLC)],
                    out_ref.at[cid, pl.ds(sid * SLC, SLC)])


def _sc_agg(src2d, dst2d, hs, zeros_nd):
    return pl.kernel(
        _sc_agg_body,
        out_type=jax.ShapeDtypeStruct((NC, NPAD, DOUT), jnp.float32),
        mesh=_sc_mesh(),
        compiler_params=pltpu.CompilerParams(use_tc_tiling_on_sc=False),
        scratch_types=[
            pltpu.VMEM((CPW, CH), jnp.int32),
            pltpu.VMEM((CPW, CH), jnp.int32),
            [pltpu.VMEM((CH, DOUT), jnp.float32) for _ in range(NBUF)],
            [pltpu.SemaphoreType.DMA for _ in range(NBUF)],
            pltpu.VMEM_SHARED((NPAD, DOUT), jnp.float32),
            pltpu.VMEM_SHARED((NPAD, DOUT), jnp.float32),
        ],
    )(src2d, dst2d, hs, zeros_nd)


def _tc_h_body(x_ref, w_ref, h_ref):
    h = lax.dot_general(x_ref[...], w_ref[...], (((0,), (0,)), ((), ())),
                        preferred_element_type=jnp.float32)
    h_ref[0:N, :] = h
    h_ref[N:NPAD, :] = jnp.zeros((NPAD - N, DOUT), jnp.float32)


def _tc_h(x, w):
    return pl.pallas_call(
        _tc_h_body,
        out_shape=jax.ShapeDtypeStruct((NPAD, DOUT), jnp.float32),
    )(x, w)


def _tc_scale_body(h_ref, degt_ref, hs_ref):
    degsum = jnp.sum(degt_ref[...], axis=1, keepdims=True) + 1.0
    dinv = lax.rsqrt(degsum)
    hs_ref[...] = h_ref[...] * dinv


def _tc_scale(h, deg_t):
    return pl.pallas_call(
        _tc_scale_body,
        out_shape=jax.ShapeDtypeStruct((NPAD, DOUT), jnp.float32),
    )(h, deg_t)


def _tc_z_body(a0_ref, a1_ref, hs_ref, degt_ref, b_ref, z_ref):
    degsum = jnp.sum(degt_ref[...], axis=1, keepdims=True) + 1.0
    dinv = lax.rsqrt(degsum)
    z_ref[...] = dinv * (a0_ref[...] + a1_ref[...] + hs_ref[...]) + b_ref[...]


def _tc_z(a0, a1, hs, deg_t, b2):
    return pl.pallas_call(
        _tc_z_body,
        out_shape=jax.ShapeDtypeStruct((NPAD, DOUT), jnp.float32),
    )(a0, a1, hs, deg_t, b2)


BM = 2048
BN = 2048


def _tc_dec_body(z_ref, zt_ref, o_ref):
    d = lax.dot_general(z_ref[...], zt_ref[...], (((1,), (0,)), ((), ())),
                        preferred_element_type=jnp.float32)
    o_ref[...] = 0.5 + 0.5 * jnp.tanh(0.5 * d)


def _tc_decode(z, zt):
    return pl.pallas_call(
        _tc_dec_body,
        grid=(NPAD // BM, NPAD // BN),
        in_specs=[
            pl.BlockSpec((BM, DOUT), lambda i, j: (i, 0)),
            pl.BlockSpec((DOUT, BN), lambda i, j: (0, j)),
        ],
        out_specs=pl.BlockSpec((BM, BN), lambda i, j: (i, j)),
        out_shape=jax.ShapeDtypeStruct((N, N), jnp.float32),
        compiler_params=pltpu.CompilerParams(
            dimension_semantics=("parallel", "parallel")),
    )(z, zt)


@jax.jit
def kernel(x, edge_index, W_enc, b_enc):
    src = edge_index[0]
    dst = edge_index[1]
    pad = EPAD - E
    src_p = jnp.concatenate([src, jnp.zeros((pad,), jnp.int32)])
    dst_p = jnp.concatenate([dst, jnp.full((pad,), NPAD - 1, jnp.int32)])
    src2d = src_p.reshape(NW * CPW, CH)
    dst2d = dst_p.reshape(NW * CPW, CH)

    zeros_n = jnp.zeros((NPAD,), jnp.float32)
    ones_ch = jnp.ones((CH,), jnp.float32)
    zeros_nd = jnp.zeros((NPAD, DOUT), jnp.float32)

    deg_p = _sc_deg(dst2d, zeros_n, ones_ch)
    deg_t = deg_p.T

    h = _tc_h(x, W_enc)
    hs = _tc_scale(h, deg_t)

    agg_p = _sc_agg(src2d, dst2d, hs, zeros_nd)

    z = _tc_z(agg_p[0], agg_p[1], hs, deg_t,
              b_enc.reshape(1, DOUT))
    adj = _tc_decode(z, z.T)
    return adj
